# baseline (speedup 1.0000x reference)
import functools

import jax
import jax.numpy as jnp
from jax import lax
from jax.experimental import pallas as pl
from jax.experimental.pallas import tpu as pltpu
from jax.experimental.pallas import tpu_sc as plsc

_EPS = 1e-7
_ROWS = 128
_COLS = 32768
_LANES = 16
_NBLK = _COLS // _LANES
_NWORKERS = 32
_ROWS_PER_W = _ROWS // _NWORKERS
_NEG_INF = float("-inf")


def _sortd(v):
    k, _ = plsc.sort_key_val(v, v, descending=True)
    return k


def _tophalf(a, b):
    return _sortd(jnp.maximum(a, lax.rev(b, (0,))))


def _bothhalves(a, b):
    br = lax.rev(b, (0,))
    return _sortd(jnp.maximum(a, br)), _sortd(jnp.minimum(a, br))


_GATHER_DN = lax.GatherDimensionNumbers(
    offset_dims=(), collapsed_slice_dims=(0,), start_index_map=(0,))


def _bcast_last(v):
    idx = jnp.full((_LANES, 1), _LANES - 1, jnp.int32)
    return lax.gather(v, idx, _GATHER_DN, (1,),
                      mode=lax.GatherScatterMode.PROMISE_IN_BOUNDS)


_SG = 8
_NSG = _NBLK // _SG


def _merge_block(x, s0, s1):
    h = _tophalf(x, s1)
    return _bothhalves(s0, h)


@functools.partial(
    pl.kernel,
    out_type=jax.ShapeDtypeStruct((_ROWS, _COLS), jnp.float32),
    mesh=plsc.VectorSubcoreMesh(core_axis_name="c", subcore_axis_name="s"),
    compiler_params=pltpu.CompilerParams(needs_layout_passes=False),
    scratch_types=[
        pltpu.VMEM((_COLS,), jnp.float32),
        pltpu.VMEM((_COLS,), jnp.float32),
        pltpu.VMEM((_COLS + _LANES,), jnp.float32),
        pltpu.VMEM((_NSG + _LANES,), jnp.float32),
        pltpu.VMEM((_NSG + _LANES,), jnp.int32),
        pltpu.SemaphoreType.DMA,
        pltpu.SemaphoreType.DMA,
        pltpu.SemaphoreType.DMA,
        pltpu.SemaphoreType.DMA,
    ],
)
def _sc_topk_norm(x_hbm, o_hbm, row_a, row_b, cand_buf, sgmax_buf, wl_buf,
                  sem_a, sem_b, sem_oa, sem_ob):
    wid = lax.axis_index("s") * 2 + lax.axis_index("c")
    lane_iota = lax.iota(jnp.int32, _LANES)
    row0 = wid * _ROWS_PER_W
    bufs = [row_a, row_b]
    sems = [sem_a, sem_b]
    osems = [sem_oa, sem_ob]
    ninf = jnp.full((_LANES,), _NEG_INF, jnp.float32)
    zero = jnp.zeros((_LANES,), jnp.int32)

    def process(row, row_buf, osem):
        def pass_a(i2, ms):
            m0, m1 = ms
            for t in range(2):
                i = i2 * 2 + t
                base = i * (_SG * _LANES)
                v = [row_buf[pl.ds(base + j * _LANES, _LANES)]
                     for j in range(_SG)]
                e = jnp.maximum(jnp.maximum(v[0], v[2]),
                                jnp.maximum(v[4], v[6]))
                o = jnp.maximum(jnp.maximum(v[1], v[3]),
                                jnp.maximum(v[5], v[7]))
                sm = plsc.cummax(jnp.maximum(e, o))
                plsc.store_scatter(sgmax_buf,
                                   [jnp.full((_LANES,), i, jnp.int32)], sm,
                                   mask=lane_iota == _LANES - 1)
                m0, m1 = jnp.maximum(m0, e), jnp.maximum(m1, o)
            return m0, m1

        m0, m1 = lax.fori_loop(0, _NSG // 2, pass_a, (ninf, ninf))
        g = _bcast_last(_sortd(jnp.minimum(m0, m1)))

        woff = zero
        for v in range(_NSG // _LANES):
            sgm = sgmax_buf[pl.ds(v * _LANES, _LANES)]
            m = sgm >= g
            pos = plsc.cumsum(jnp.where(m, 1, 0).astype(jnp.int32))
            plsc.store_scatter(wl_buf, [woff + pos - 1],
                               lane_iota + v * _LANES, mask=m)
            woff = woff + plsc.all_reduce_population_count(m)
        n_sg = jnp.max(woff)

        def pass_b(i, st):
            coff, base = st
            nbase = wl_buf[pl.ds(i + 1, _LANES)][0] * (_SG * _LANES)
            for j in range(_SG):
                x = row_buf[pl.ds(base + j * _LANES, _LANES)]
                m = x >= g
                pos = plsc.cumsum(jnp.where(m, 1, 0).astype(jnp.int32))
                plsc.store_scatter(cand_buf, [coff + pos - 1], x, mask=m)
                coff = coff + plsc.all_reduce_population_count(m)
            return coff, nbase

        base0 = wl_buf[pl.ds(0, _LANES)][0] * (_SG * _LANES)
        coff, _ = lax.fori_loop(0, n_sg, pass_b, (zero, base0))
        ncand = jnp.max(coff)
        nfull = ncand // _LANES

        def pass_c(k, st):
            s0, s1 = st
            x = _sortd(cand_buf[pl.ds(k * _LANES, _LANES)])
            return _merge_block(x, s0, s1)

        s0, s1 = lax.fori_loop(0, nfull, pass_c, (ninf, ninf))
        xt = cand_buf[pl.ds(nfull * _LANES, _LANES)]
        xt = jnp.where(lane_iota < (ncand - nfull * _LANES), xt, ninf)
        s0, s1 = _merge_block(_sortd(xt), s0, s1)

        dv = _bcast_last(s1) + jnp.full((_LANES,), _EPS, jnp.float32)

        def sum_c(k, acc):
            c = cand_buf[pl.ds(k * _LANES, _LANES)]
            return acc + jnp.maximum(c - dv, 0.0)

        acc = lax.fori_loop(0, nfull, sum_c,
                            jnp.zeros((_LANES,), jnp.float32))
        acc = acc + jnp.maximum(xt - dv, 0.0)
        tot = _bcast_last(plsc.cumsum(acc))
        inv = jnp.full((_LANES,), 1.0, jnp.float32) / (
            tot + jnp.full((_LANES,), _EPS, jnp.float32))

        def pass_d(i, carry):
            base = i * (_SG * _LANES)
            for j in range(_SG):
                sl = pl.ds(base + j * _LANES, _LANES)
                row_buf[sl] = jnp.maximum(row_buf[sl] - dv, 0.0) * inv
            return carry

        lax.fori_loop(0, _NSG, pass_d, 0)
        return pltpu.async_copy(row_buf, o_hbm.at[row], osem)

    out_cps = [None, None]
    cp = pltpu.async_copy(x_hbm.at[row0], bufs[0], sems[0])
    for r in range(_ROWS_PER_W):
        nxt = None
        if r + 1 < _ROWS_PER_W:
            b = (r + 1) % 2
            if out_cps[b] is not None:
                out_cps[b].wait()
                out_cps[b] = None
            nxt = pltpu.async_copy(x_hbm.at[row0 + r + 1], bufs[b], sems[b])
        cp.wait()
        out_cps[r % 2] = process(row0 + r, bufs[r % 2], osems[r % 2])
        cp = nxt
    for ocp in out_cps:
        if ocp is not None:
            ocp.wait()


def kernel(attn_raw):
    return _sc_topk_norm(attn_raw)


# speedup vs baseline: 1.0034x; 1.0034x over previous
import functools

import jax
import jax.numpy as jnp
from jax import lax
from jax.experimental import pallas as pl
from jax.experimental.pallas import tpu as pltpu
from jax.experimental.pallas import tpu_sc as plsc

_EPS = 1e-7
_ROWS = 128
_COLS = 32768
_LANES = 16
_NBLK = _COLS // _LANES
_NWORKERS = 32
_ROWS_PER_W = _ROWS // _NWORKERS
_NEG_INF = float("-inf")


def _sortd(v):
    k, _ = plsc.sort_key_val(v, v, descending=True)
    return k


def _tophalf(a, b):
    return _sortd(jnp.maximum(a, lax.rev(b, (0,))))


def _bothhalves(a, b):
    br = lax.rev(b, (0,))
    return _sortd(jnp.maximum(a, br)), _sortd(jnp.minimum(a, br))


_GATHER_DN = lax.GatherDimensionNumbers(
    offset_dims=(), collapsed_slice_dims=(0,), start_index_map=(0,))


def _bcast_last(v):
    idx = jnp.full((_LANES, 1), _LANES - 1, jnp.int32)
    return lax.gather(v, idx, _GATHER_DN, (1,),
                      mode=lax.GatherScatterMode.PROMISE_IN_BOUNDS)


_SG = 8
_NSG = _NBLK // _SG


def _merge_block(x, s0, s1):
    h = _tophalf(x, s1)
    return _bothhalves(s0, h)


@functools.partial(
    pl.kernel,
    out_type=jax.ShapeDtypeStruct((_ROWS, _COLS), jnp.float32),
    mesh=plsc.VectorSubcoreMesh(core_axis_name="c", subcore_axis_name="s"),
    compiler_params=pltpu.CompilerParams(needs_layout_passes=False),
    scratch_types=[
        pltpu.VMEM((_COLS,), jnp.float32),
        pltpu.VMEM((_COLS,), jnp.float32),
        pltpu.VMEM((_COLS + _LANES,), jnp.float32),
        pltpu.VMEM((_NSG + _LANES,), jnp.float32),
        pltpu.VMEM((_NSG + _LANES,), jnp.int32),
        pltpu.SemaphoreType.DMA,
        pltpu.SemaphoreType.DMA,
        pltpu.SemaphoreType.DMA,
        pltpu.SemaphoreType.DMA,
    ],
)
def _sc_topk_norm(x_hbm, o_hbm, row_a, row_b, cand_buf, sgmax_buf, wl_buf,
                  sem_a, sem_b, sem_oa, sem_ob):
    wid = lax.axis_index("s") * 2 + lax.axis_index("c")
    lane_iota = lax.iota(jnp.int32, _LANES)
    row0 = wid * _ROWS_PER_W
    bufs = [row_a, row_b]
    sems = [sem_a, sem_b]
    osems = [sem_oa, sem_ob]
    ninf = jnp.full((_LANES,), _NEG_INF, jnp.float32)
    zero = jnp.zeros((_LANES,), jnp.int32)

    def process(row, row_buf, osem):
        def pass_a(i, ms):
            m0, m1 = ms
            base = i * (_SG * _LANES)
            v = [row_buf[pl.ds(base + j * _LANES, _LANES)]
                 for j in range(_SG)]
            e = jnp.maximum(jnp.maximum(v[0], v[2]), jnp.maximum(v[4], v[6]))
            o = jnp.maximum(jnp.maximum(v[1], v[3]), jnp.maximum(v[5], v[7]))
            sm = plsc.cummax(jnp.maximum(e, o))
            plsc.store_scatter(sgmax_buf,
                               [jnp.full((_LANES,), i, jnp.int32)], sm,
                               mask=lane_iota == _LANES - 1)
            return jnp.maximum(m0, e), jnp.maximum(m1, o)

        m0, m1 = lax.fori_loop(0, _NSG, pass_a, (ninf, ninf))
        g = _bcast_last(_sortd(jnp.minimum(m0, m1)))

        woff = zero
        for v in range(_NSG // _LANES):
            sgm = sgmax_buf[pl.ds(v * _LANES, _LANES)]
            m = sgm >= g
            pos = plsc.cumsum(jnp.where(m, 1, 0).astype(jnp.int32))
            plsc.store_scatter(wl_buf, [woff + pos - 1],
                               lane_iota + v * _LANES, mask=m)
            woff = woff + plsc.all_reduce_population_count(m)
        n_sg = jnp.max(woff)

        def pass_b(i, st):
            coff, base = st
            nbase = wl_buf[pl.ds(i + 1, _LANES)][0] * (_SG * _LANES)
            for j in range(_SG):
                x = row_buf[pl.ds(base + j * _LANES, _LANES)]
                m = x >= g
                pos = plsc.cumsum(jnp.where(m, 1, 0).astype(jnp.int32))
                plsc.store_scatter(cand_buf, [coff + pos - 1], x, mask=m)
                coff = coff + plsc.all_reduce_population_count(m)
            return coff, nbase

        base0 = wl_buf[pl.ds(0, _LANES)][0] * (_SG * _LANES)
        coff, _ = lax.fori_loop(0, n_sg, pass_b, (zero, base0))
        ncand = jnp.max(coff)
        nfull = ncand // _LANES

        def pass_c(k, st):
            s0, s1 = st
            x = _sortd(cand_buf[pl.ds(k * _LANES, _LANES)])
            return _merge_block(x, s0, s1)

        s0, s1 = lax.fori_loop(0, nfull, pass_c, (ninf, ninf))
        xt = cand_buf[pl.ds(nfull * _LANES, _LANES)]
        xt = jnp.where(lane_iota < (ncand - nfull * _LANES), xt, ninf)
        s0, s1 = _merge_block(_sortd(xt), s0, s1)

        dv = _bcast_last(s1) + jnp.full((_LANES,), _EPS, jnp.float32)

        def sum_c(k, acc):
            c = cand_buf[pl.ds(k * _LANES, _LANES)]
            return acc + jnp.maximum(c - dv, 0.0)

        acc = lax.fori_loop(0, nfull, sum_c,
                            jnp.zeros((_LANES,), jnp.float32))
        acc = acc + jnp.maximum(xt - dv, 0.0)
        tot = _bcast_last(plsc.cumsum(acc))
        inv = jnp.full((_LANES,), 1.0, jnp.float32) / (
            tot + jnp.full((_LANES,), _EPS, jnp.float32))

        def pass_d(i, carry):
            base = i * (_SG * _LANES)
            for j in range(_SG):
                sl = pl.ds(base + j * _LANES, _LANES)
                row_buf[sl] = jnp.maximum(row_buf[sl] - dv, 0.0) * inv
            return carry

        lax.fori_loop(0, _NSG, pass_d, 0)
        return pltpu.async_copy(row_buf, o_hbm.at[row], osem)

    out_cps = [None, None]
    cp = pltpu.async_copy(x_hbm.at[row0], bufs[0], sems[0])
    for r in range(_ROWS_PER_W):
        nxt = None
        if r + 1 < _ROWS_PER_W:
            b = (r + 1) % 2
            if out_cps[b] is not None:
                out_cps[b].wait()
                out_cps[b] = None
            nxt = pltpu.async_copy(x_hbm.at[row0 + r + 1], bufs[b], sems[b])
        cp.wait()
        out_cps[r % 2] = process(row0 + r, bufs[r % 2], osems[r % 2])
        cp = nxt
    for ocp in out_cps:
        if ocp is not None:
            ocp.wait()


def kernel(attn_raw):
    return _sc_topk_norm(attn_raw)
